# baseline (speedup 1.0000x reference)
import functools

import jax
import jax.numpy as jnp
from jax import lax
from jax.experimental import pallas as pl
from jax.experimental.pallas import tpu as pltpu
from jax.experimental.pallas import tpu_sc as plsc

NC = 2
NS = 16
L = 16
CHUNK = 128


def kernel(users, items, user_latent, item_latent, user_bias, item_bias):
    B = users.shape[0]
    D = user_latent.shape[1]
    NW = NC * NS
    b_per_w = B // NW
    n_chunks = b_per_w // CHUNK

    mesh = plsc.VectorSubcoreMesh(core_axis_name="c", subcore_axis_name="s")

    @functools.partial(
        pl.kernel,
        out_type=jax.ShapeDtypeStruct((B,), jnp.float32),
        mesh=mesh,
        compiler_params=pltpu.CompilerParams(
            needs_layout_passes=False, use_tc_tiling_on_sc=False),
        scratch_types=[
            pltpu.VMEM((b_per_w,), jnp.int32),
            pltpu.VMEM((b_per_w,), jnp.int32),
            pltpu.VMEM((b_per_w, D), jnp.float32),
            pltpu.VMEM((b_per_w, D), jnp.float32),
            pltpu.VMEM((b_per_w,), jnp.float32),
            pltpu.VMEM((b_per_w,), jnp.float32),
            pltpu.VMEM((b_per_w,), jnp.float32),
            pltpu.SemaphoreType.DMA,
        ],
    )
    def run(users_hbm, items_hbm, ul_hbm, il_hbm, ub_hbm, ib_hbm, out_hbm,
            idx_u, idx_i, u_rows, i_rows, ub_v, ib_v, out_v, sem):
        wid = lax.axis_index("s") * NC + lax.axis_index("c")
        base = wid * b_per_w

        pltpu.sync_copy(users_hbm.at[pl.ds(base, b_per_w)], idx_u)
        pltpu.sync_copy(items_hbm.at[pl.ds(base, b_per_w)], idx_i)

        copies = []
        for c in range(n_chunks):
            s = pl.ds(c * CHUNK, CHUNK)
            copies.append(pltpu.async_copy(ul_hbm.at[idx_u.at[s]], u_rows.at[s], sem))
            copies.append(pltpu.async_copy(il_hbm.at[idx_i.at[s]], i_rows.at[s], sem))
            copies.append(pltpu.async_copy(ub_hbm.at[idx_u.at[s]], ub_v.at[s], sem))
            copies.append(pltpu.async_copy(ib_hbm.at[idx_i.at[s]], ib_v.at[s], sem))
        for cp in copies:
            cp.wait()

        def group_body(g, carry):
            rows = g * L + lax.iota(jnp.int32, L)
            acc = plsc.load_gather(ub_v, [rows]) + plsc.load_gather(ib_v, [rows])
            for d in range(D):
                col = jnp.full((L,), d, jnp.int32)
                acc = acc + (plsc.load_gather(u_rows, [rows, col]) *
                             plsc.load_gather(i_rows, [rows, col]))
            out_v[pl.ds(g * L, L)] = acc
            return carry

        lax.fori_loop(0, b_per_w // L, group_body, 0)

        pltpu.sync_copy(out_v, out_hbm.at[pl.ds(base, b_per_w)])

    return run(users.astype(jnp.int32), items.astype(jnp.int32),
               user_latent, item_latent,
               user_bias.reshape(-1), item_bias.reshape(-1))
